# baseline (speedup 1.0000x reference)
import functools

import jax
import jax.numpy as jnp
from jax import lax
from jax.experimental import pallas as pl
from jax.experimental.pallas import tpu as pltpu
from jax.experimental.pallas import tpu_sc as plsc

_B, _C, _H, _W = 16, 3, 512, 512
_PLANE = _H * _W
_NTOT = 16 ** 3
_ROWS = 8
_NCHUNK = (_H // 2) // _ROWS
_L = 16
_UNROLL = 8


_mesh = plsc.VectorSubcoreMesh(core_axis_name="c", subcore_axis_name="s")


@functools.partial(
    pl.kernel,
    out_type=jax.ShapeDtypeStruct((2, _B, _NTOT), jnp.float32),
    mesh=_mesh,
    scratch_types=[
        pltpu.VMEM((_ROWS, _W), jnp.float32),
        pltpu.VMEM((_ROWS, _W), jnp.float32),
        pltpu.VMEM((_ROWS, _W), jnp.float32),
        pltpu.VMEM((_ROWS, _W), jnp.float32),
        pltpu.VMEM((_ROWS, _W), jnp.float32),
        pltpu.VMEM((_ROWS, _W), jnp.float32),
        pltpu.VMEM((_NTOT,), jnp.float32),
        pltpu.SemaphoreType.DMA,
        pltpu.SemaphoreType.DMA,
    ],
    compiler_params=pltpu.CompilerParams(needs_layout_passes=False),
)
def _hist_sc(img_hbm, out_hbm, b00, b01, b02, b10, b11, b12, hist, sem0, sem1):
    bufs = ((b00, b01, b02), (b10, b11, b12))
    nc = _mesh.num_cores
    wid = lax.axis_index("s") * nc + lax.axis_index("c")
    img = wid // 2
    half = wid % 2
    row_base = half * (_H // 2)
    sems = (sem0, sem1)

    def start(k, slot):
        kc = jnp.minimum(k, _NCHUNK - 1)
        r0 = row_base + kc * _ROWS
        for c in range(_C):
            pltpu.async_copy(
                img_hbm.at[img, c, pl.ds(r0, _ROWS), :],
                bufs[slot][c],
                sems[slot],
            )

    def wait(slot):
        for c in range(_C):
            pltpu.make_async_copy(
                img_hbm.at[img, c, pl.ds(row_base, _ROWS), :],
                bufs[slot][c],
                sems[slot],
            ).wait()

    ones = jnp.ones((_L,), jnp.float32)

    vregs_per_row = _W // (_L * _UNROLL)

    def process(slot):
        def body(v, carry):
            row = v // vregs_per_row
            base_col = (v % vregs_per_row) * (_L * _UNROLL)
            vals = [
                bufs[slot][c][row, pl.ds(base_col + u * _L, _L)]
                for c in range(_C)
                for u in range(_UNROLL)
            ]
            t = [x * 8.0 for x in vals]
            t = [x + 8.0 for x in t]
            t = [jnp.maximum(x, 0.0) for x in t]
            t = [jnp.minimum(x, 15.0) for x in t]
            t = [x.astype(jnp.int32) for x in t]
            binvs = [
                t[u] | (t[_UNROLL + u] << 4) | (t[2 * _UNROLL + u] << 8)
                for u in range(_UNROLL)
            ]
            for u in range(_UNROLL):
                plsc.addupdate_scatter(hist, [binvs[u]], ones)
            return carry

        lax.fori_loop(0, _ROWS * vregs_per_row, body, 0)

    start(jnp.int32(0), 0)
    start(jnp.int32(1), 1)

    def zero_body(i, carry):
        hist[pl.ds(i * _L, _L)] = jnp.zeros((_L,), jnp.float32)
        return carry

    lax.fori_loop(0, _NTOT // _L, zero_body, 0)

    def ring_body(j, carry):
        k = j * 2
        wait(0)
        process(0)
        start(k + 2, 0)
        wait(1)
        process(1)
        start(k + 3, 1)
        return carry

    lax.fori_loop(0, _NCHUNK // 2, ring_body, 0)

    wait(0)
    wait(1)

    pltpu.sync_copy(hist, out_hbm.at[half, img])


def _loss_body(p_ref, t_ref, o_ref):
    h = (p_ref[0] + p_ref[1]) * (1.0 / _PLANE)
    diff = jnp.abs(h - t_ref[...])
    o_ref[...] = jnp.sum(diff).reshape(1, 1) * (1.0 / (_B * _NTOT))


_loss_tc = pl.pallas_call(
    _loss_body,
    out_shape=jax.ShapeDtypeStruct((1, 1), jnp.float32),
)


@jax.jit
def kernel(input, color_bins, bin_scale, target):
    partials = _hist_sc(input)
    h = (partials[0] + partials[1]) * (1.0 / _PLANE)
    return jnp.mean(jnp.abs(h - target[None, :]))


# speedup vs baseline: 1.1798x; 1.1798x over previous
import functools

import jax
import jax.numpy as jnp
from jax import lax
from jax.experimental import pallas as pl
from jax.experimental.pallas import tpu as pltpu
from jax.experimental.pallas import tpu_sc as plsc

_B, _C, _H, _W = 16, 3, 512, 512
_PLANE = _H * _W
_NTOT = 16 ** 3
_ROWS = 8
_NCHUNK = (_H // 2) // _ROWS
_L = 16
_UNROLL = 8


_mesh = plsc.VectorSubcoreMesh(core_axis_name="c", subcore_axis_name="s")


@functools.partial(
    pl.kernel,
    out_type=jax.ShapeDtypeStruct((2, _B, _NTOT), jnp.float32),
    mesh=_mesh,
    scratch_types=[
        pltpu.VMEM((_ROWS, _W), jnp.float32),
        pltpu.VMEM((_ROWS, _W), jnp.float32),
        pltpu.VMEM((_ROWS, _W), jnp.float32),
        pltpu.VMEM((_ROWS, _W), jnp.float32),
        pltpu.VMEM((_ROWS, _W), jnp.float32),
        pltpu.VMEM((_ROWS, _W), jnp.float32),
        pltpu.VMEM((_NTOT,), jnp.float32),
        pltpu.SemaphoreType.DMA,
        pltpu.SemaphoreType.DMA,
    ],
    compiler_params=pltpu.CompilerParams(needs_layout_passes=False),
)
def _hist_sc(img_hbm, out_hbm, b00, b01, b02, b10, b11, b12, hist, sem0, sem1):
    bufs = ((b00, b01, b02), (b10, b11, b12))
    nc = _mesh.num_cores
    wid = lax.axis_index("s") * nc + lax.axis_index("c")
    img = wid // 2
    half = wid % 2
    row_base = half * (_H // 2)
    sems = (sem0, sem1)

    def start(k, slot):
        kc = jnp.minimum(k, _NCHUNK - 1)
        r0 = row_base + kc * _ROWS
        for c in range(_C):
            pltpu.async_copy(
                img_hbm.at[img, c, pl.ds(r0, _ROWS), :],
                bufs[slot][c],
                sems[slot],
            )

    def wait(slot):
        for c in range(_C):
            pltpu.make_async_copy(
                img_hbm.at[img, c, pl.ds(row_base, _ROWS), :],
                bufs[slot][c],
                sems[slot],
            ).wait()

    ones = jnp.ones((_L,), jnp.float32)

    vregs_per_row = _W // (_L * _UNROLL)

    def process(slot):
        @plsc.parallel_loop(0, _ROWS * vregs_per_row, unroll=2)
        def body(v):
            row = v // vregs_per_row
            base_col = (v % vregs_per_row) * (_L * _UNROLL)
            vals = [
                bufs[slot][c][row, pl.ds(base_col + u * _L, _L)]
                for c in range(_C)
                for u in range(_UNROLL)
            ]
            t = [x * 8.0 for x in vals]
            t = [x + 8.0 for x in t]
            t = [jnp.maximum(x, 0.0) for x in t]
            t = [jnp.minimum(x, 15.0) for x in t]
            t = [x.astype(jnp.int32) for x in t]
            binvs = [
                t[u] | (t[_UNROLL + u] << 4) | (t[2 * _UNROLL + u] << 8)
                for u in range(_UNROLL)
            ]
            for u in range(_UNROLL):
                plsc.addupdate_scatter(hist, [binvs[u]], ones)

    start(jnp.int32(0), 0)
    start(jnp.int32(1), 1)

    def zero_body(i, carry):
        hist[pl.ds(i * _L, _L)] = jnp.zeros((_L,), jnp.float32)
        return carry

    lax.fori_loop(0, _NTOT // _L, zero_body, 0)

    def ring_body(j, carry):
        k = j * 2
        wait(0)
        process(0)
        start(k + 2, 0)
        wait(1)
        process(1)
        start(k + 3, 1)
        return carry

    lax.fori_loop(0, _NCHUNK // 2, ring_body, 0)

    wait(0)
    wait(1)

    pltpu.sync_copy(hist, out_hbm.at[half, img])


def _loss_body(p_ref, t_ref, o_ref):
    h = (p_ref[0] + p_ref[1]) * (1.0 / _PLANE)
    diff = jnp.abs(h - t_ref[...])
    o_ref[...] = jnp.sum(diff).reshape(1, 1) * (1.0 / (_B * _NTOT))


_loss_tc = pl.pallas_call(
    _loss_body,
    out_shape=jax.ShapeDtypeStruct((1, 1), jnp.float32),
)


@jax.jit
def kernel(input, color_bins, bin_scale, target):
    partials = _hist_sc(input)
    loss = _loss_tc(partials, target.reshape(1, _NTOT))
    return loss[0, 0]


# speedup vs baseline: 1.2462x; 1.0563x over previous
import functools

import jax
import jax.numpy as jnp
from jax import lax
from jax.experimental import pallas as pl
from jax.experimental.pallas import tpu as pltpu
from jax.experimental.pallas import tpu_sc as plsc

_B, _C, _H, _W = 16, 3, 512, 512
_PLANE = _H * _W
_NTOT = 16 ** 3
_ROWS = 8
_NCHUNK = (_H // 2) // _ROWS
_L = 16
_UNROLL = 8


_mesh = plsc.VectorSubcoreMesh(core_axis_name="c", subcore_axis_name="s")


@functools.partial(
    pl.kernel,
    out_type=jax.ShapeDtypeStruct((2, _B, _NTOT), jnp.float32),
    mesh=_mesh,
    scratch_types=[
        pltpu.VMEM((_ROWS, _W), jnp.float32),
        pltpu.VMEM((_ROWS, _W), jnp.float32),
        pltpu.VMEM((_ROWS, _W), jnp.float32),
        pltpu.VMEM((_ROWS, _W), jnp.float32),
        pltpu.VMEM((_ROWS, _W), jnp.float32),
        pltpu.VMEM((_ROWS, _W), jnp.float32),
        pltpu.VMEM((_NTOT,), jnp.float32),
        pltpu.SemaphoreType.DMA,
        pltpu.SemaphoreType.DMA,
    ],
    compiler_params=pltpu.CompilerParams(needs_layout_passes=False),
)
def _hist_sc(img_hbm, out_hbm, b00, b01, b02, b10, b11, b12, hist, sem0, sem1):
    bufs = ((b00, b01, b02), (b10, b11, b12))
    nc = _mesh.num_cores
    wid = lax.axis_index("s") * nc + lax.axis_index("c")
    img = wid // 2
    half = wid % 2
    row_base = half * (_H // 2)
    sems = (sem0, sem1)

    def start(k, slot):
        kc = jnp.minimum(k, _NCHUNK - 1)
        r0 = row_base + kc * _ROWS
        for c in range(_C):
            pltpu.async_copy(
                img_hbm.at[img, c, pl.ds(r0, _ROWS), :],
                bufs[slot][c],
                sems[slot],
            )

    def wait(slot):
        for c in range(_C):
            pltpu.make_async_copy(
                img_hbm.at[img, c, pl.ds(row_base, _ROWS), :],
                bufs[slot][c],
                sems[slot],
            ).wait()

    ones = jnp.ones((_L,), jnp.float32)

    vregs_per_row = _W // (_L * _UNROLL)

    def process(slot):
        @plsc.parallel_loop(0, _ROWS * vregs_per_row, unroll=2)
        def body(v):
            row = v // vregs_per_row
            base_col = (v % vregs_per_row) * (_L * _UNROLL)
            vals = [
                bufs[slot][c][row, pl.ds(base_col + u * _L, _L)]
                for c in range(_C)
                for u in range(_UNROLL)
            ]
            t = [x * 8.0 for x in vals]
            t = [x + 23.5 for x in t]
            t = [jnp.maximum(x, 16.0) for x in t]
            t = [jnp.minimum(x, 31.0) for x in t]
            t = [x + 8388608.0 for x in t]
            t = [plsc.bitcast(x, jnp.int32) for x in t]
            binvs = [
                t[u] + (t[_UNROLL + u] << 4) + (t[2 * _UNROLL + u] << 8)
                + jnp.int32(0x04FFEEF0)
                for u in range(_UNROLL)
            ]
            for u in range(_UNROLL):
                plsc.addupdate_scatter(hist, [binvs[u]], ones)

    start(jnp.int32(0), 0)
    start(jnp.int32(1), 1)

    def zero_body(i, carry):
        hist[pl.ds(i * _L, _L)] = jnp.zeros((_L,), jnp.float32)
        return carry

    lax.fori_loop(0, _NTOT // _L, zero_body, 0)

    def ring_body(j, carry):
        k = j * 2
        wait(0)
        process(0)
        start(k + 2, 0)
        wait(1)
        process(1)
        start(k + 3, 1)
        return carry

    lax.fori_loop(0, _NCHUNK // 2, ring_body, 0)

    wait(0)
    wait(1)

    pltpu.sync_copy(hist, out_hbm.at[half, img])


def _loss_body(p_ref, t_ref, o_ref):
    h = (p_ref[0] + p_ref[1]) * (1.0 / _PLANE)
    diff = jnp.abs(h - t_ref[...])
    o_ref[...] = jnp.sum(diff).reshape(1, 1) * (1.0 / (_B * _NTOT))


_loss_tc = pl.pallas_call(
    _loss_body,
    out_shape=jax.ShapeDtypeStruct((1, 1), jnp.float32),
)


@jax.jit
def kernel(input, color_bins, bin_scale, target):
    partials = _hist_sc(input)
    loss = _loss_tc(partials, target.reshape(1, _NTOT))
    return loss[0, 0]
